# baseline (speedup 1.0000x reference)
import functools

import jax
import jax.numpy as jnp
from jax import lax
from jax.experimental import pallas as pl
from jax.experimental.pallas import tpu as pltpu
from jax.experimental.pallas import tpu_sc as plsc

_NUM_CORES = 2
_NUM_SUBCORES = 16
_LANES = 16
_NW = _NUM_CORES * _NUM_SUBCORES

_H_SC = 96


def _sqrt_vec(x):
    i = plsc.bitcast(x, jnp.int32)
    i = jnp.full((_LANES,), 0x5F3759DF, jnp.int32) - (i >> 1)
    r = plsc.bitcast(i, jnp.float32)
    xh = x * 0.5
    r = r * (1.5 - xh * r * r)
    r = r * (1.5 - xh * r * r)
    return x * r


@functools.lru_cache(maxsize=None)
def _sc_chamfer(h, w, n, h_sc):
    rows_w = h_sc // _NW
    groups = w // _LANES
    pts_w = n // _NW
    assert rows_w * _NW == h_sc and groups * _LANES == w and pts_w == _LANES

    mesh = plsc.VectorSubcoreMesh(core_axis_name="c", subcore_axis_name="s")

    @functools.partial(
        pl.kernel,
        out_type=jax.ShapeDtypeStruct((_NW, _LANES), jnp.float32),
        mesh=mesh,
        compiler_params=pltpu.CompilerParams(needs_layout_passes=False),
        scratch_types=[
            pltpu.VMEM((n,), jnp.float32),
            pltpu.VMEM((n,), jnp.float32),
            pltpu.VMEM((_LANES,), jnp.float32),
        ],
    )
    def sc_k(py_hbm, px_hbm, out, py_v, px_v, part_v):
        cid = lax.axis_index("c")
        sid = lax.axis_index("s")
        wid = sid * _NUM_CORES + cid

        pltpu.sync_copy(py_hbm, py_v)
        pltpu.sync_copy(px_hbm, px_v)

        p0 = wid * pts_w
        pyv = py_v[pl.ds(p0, _LANES)]
        pxv = px_v[pl.ds(p0, _LANES)]

        def lattice_d2(v, hi):
            idx = (v + 0.5).astype(jnp.int32)
            idx = jnp.minimum(jnp.maximum(idx, 0), hi)
            d = v - idx.astype(jnp.float32)
            return d * d

        sum_vec = _sqrt_vec(lattice_d2(pyv, h - 1) + lattice_d2(pxv, w - 1))

        row0 = wid * rows_w
        iota = lax.iota(jnp.int32, _LANES).astype(jnp.float32)
        yfs = [(row0 + r).astype(jnp.float32) for r in range(rows_w)]
        chunks = n // _LANES

        tile_w = 8
        g0 = 0
        while g0 < groups:
            gw = min(tile_w, groups - g0)
            assert gw % 2 == 0
            xv_t = [iota + float((g0 + g) * _LANES) for g in range(gw)]
            big = jnp.full((_LANES,), 1e30, jnp.float32)
            bigb = plsc.pack(big, big, format=plsc.PackFormat.INTERLEAVED)
            init = tuple(bigb for _ in range(rows_w * (gw // 2)))

            def body(jc, accs, xv_t=xv_t, gw=gw):
                j0 = jc * _LANES
                pyv = py_v[pl.ds(j0, _LANES)]
                pxv = px_v[pl.ds(j0, _LANES)]
                out = list(accs)
                for k in range(_LANES):
                    pyj = pyv[k]
                    pxj = pxv[k]
                    packed = []
                    for g in range(gw // 2):
                        dxa = xv_t[2 * g] - pxj
                        dxb = xv_t[2 * g + 1] - pxj
                        packed.append(plsc.pack(dxa * dxa, dxb * dxb,
                                                format=plsc.PackFormat.INTERLEAVED))
                    for r in range(rows_w):
                        dy = yfs[r] - pyj
                        dy2v = jnp.broadcast_to(dy * dy, (_LANES,))
                        dy2b = plsc.pack(dy2v, dy2v,
                                         format=plsc.PackFormat.INTERLEAVED)
                        for g in range(gw // 2):
                            i = r * (gw // 2) + g
                            out[i] = jnp.minimum(out[i], packed[g] + dy2b)
                return tuple(out)

            accs = lax.fori_loop(0, chunks, body, init)
            for a in accs:
                fa, fb = plsc.unpack(a, format=plsc.PackFormat.INTERLEAVED,
                                     preferred_element_type=jnp.float32)
                sum_vec = sum_vec + _sqrt_vec(fa) + _sqrt_vec(fb)
            g0 += gw

        total = jnp.sum(sum_vec)
        part_v[...] = jnp.broadcast_to(total, (_LANES,))
        pltpu.sync_copy(part_v, out.at[wid])

    return sc_k


_STRIP = 1792


def _tc_band(coords_ref, p3_ref, out_ref):
    g = jnp.dot(coords_ref[...], p3_ref[...],
                preferred_element_type=jnp.float32,
                precision=lax.Precision.HIGHEST)
    d2 = jnp.maximum(jnp.min(g, axis=1), 0.0)
    s = jnp.broadcast_to(jnp.sum(jnp.sqrt(d2)), (1, 1))

    @pl.when(pl.program_id(0) == 0)
    def _init():
        out_ref[...] = jnp.zeros_like(out_ref)

    out_ref[...] = out_ref[...] + s


def kernel(img_render_points, img_ref):
    pts = img_render_points.reshape(-1, 2).astype(jnp.float32)
    n = pts.shape[0]
    h, w = img_ref.shape
    py = pts[:, 0]
    px = pts[:, 1]

    sc_partials = _sc_chamfer(h, w, n, _H_SC)(py, px)

    cy0 = 0.5 * (h - 1)
    cx0 = 0.5 * (w - 1)
    pyc = py - cy0
    pxc = px - cx0
    zs = jnp.zeros_like(pyc)
    p3 = jnp.stack([-2.0 * pyc, -2.0 * pxc, pyc * pyc + pxc * pxc,
                    jnp.ones_like(pyc), zs, zs, zs, zs], axis=0)
    ys = jnp.arange(_H_SC, h, dtype=jnp.float32) - cy0
    xs = jnp.arange(w, dtype=jnp.float32) - cx0
    cyg = jnp.repeat(ys, w)
    cxg = jnp.tile(xs, h - _H_SC)
    zg = jnp.zeros_like(cyg)
    coords = jnp.stack([cyg, cxg, jnp.ones_like(cyg),
                        cyg * cyg + cxg * cxg, zg, zg, zg, zg], axis=1)

    m = coords.shape[0]
    n_strips = m // _STRIP
    assert n_strips * _STRIP == m
    tc_out = pl.pallas_call(
        _tc_band,
        grid=(n_strips,),
        in_specs=[
            pl.BlockSpec((_STRIP, 8), lambda i: (i, 0)),
            pl.BlockSpec((8, n), lambda i: (0, 0)),
        ],
        out_specs=pl.BlockSpec((1, 1), lambda i: (0, 0)),
        out_shape=jax.ShapeDtypeStruct((1, 1), jnp.float32),
    )(coords, p3)

    return jnp.sum(sc_partials[:, 0]) + tc_out[0, 0]


# speedup vs baseline: 2.3646x; 2.3646x over previous
import functools

import jax
import jax.numpy as jnp
from jax import lax
from jax.experimental import pallas as pl
from jax.experimental.pallas import tpu as pltpu
from jax.experimental.pallas import tpu_sc as plsc

_NUM_CORES = 2
_NUM_SUBCORES = 16
_LANES = 16
_NW = _NUM_CORES * _NUM_SUBCORES

_H_SC = 96


def _sqrt_vec(x):
    i = plsc.bitcast(x, jnp.int32)
    i = jnp.full((_LANES,), 0x5F3759DF, jnp.int32) - (i >> 1)
    r = plsc.bitcast(i, jnp.float32)
    xh = x * 0.5
    r = r * (1.5 - xh * r * r)
    r = r * (1.5 - xh * r * r)
    return x * r


@functools.lru_cache(maxsize=None)
def _sc_chamfer(h, w, n, h_sc):
    rows_w = h_sc // _NW
    groups = w // _LANES
    pts_w = n // _NW
    assert rows_w * _NW == h_sc and groups * _LANES == w and pts_w == _LANES

    mesh = plsc.VectorSubcoreMesh(core_axis_name="c", subcore_axis_name="s")

    @functools.partial(
        pl.kernel,
        out_type=jax.ShapeDtypeStruct((_NW, _LANES), jnp.float32),
        mesh=mesh,
        compiler_params=pltpu.CompilerParams(needs_layout_passes=False),
        scratch_types=[
            pltpu.VMEM((n,), jnp.float32),
            pltpu.VMEM((n,), jnp.float32),
            pltpu.VMEM((_LANES,), jnp.float32),
        ],
    )
    def sc_k(py_hbm, px_hbm, out, py_v, px_v, part_v):
        cid = lax.axis_index("c")
        sid = lax.axis_index("s")
        wid = sid * _NUM_CORES + cid

        pltpu.sync_copy(py_hbm, py_v)
        pltpu.sync_copy(px_hbm, px_v)

        p0 = wid * pts_w
        pyv = py_v[pl.ds(p0, _LANES)]
        pxv = px_v[pl.ds(p0, _LANES)]

        def lattice_d2(v, hi):
            idx = (v + 0.5).astype(jnp.int32)
            idx = jnp.minimum(jnp.maximum(idx, 0), hi)
            d = v - idx.astype(jnp.float32)
            return d * d

        sum_vec = _sqrt_vec(lattice_d2(pyv, h - 1) + lattice_d2(pxv, w - 1))

        row0 = wid * rows_w
        iota = lax.iota(jnp.int32, _LANES).astype(jnp.float32)
        yfs = [(row0 + r).astype(jnp.float32) for r in range(rows_w)]
        chunks = n // _LANES

        tile_w = 14 if rows_w <= 4 else 8
        g0 = 0
        while g0 < groups:
            gw = min(tile_w, groups - g0)
            assert gw % 2 == 0
            xv_t = [iota + float((g0 + g) * _LANES) for g in range(gw)]
            big = jnp.full((_LANES,), 1e30, jnp.float32)
            bigb = plsc.pack(big, big, format=plsc.PackFormat.INTERLEAVED)
            init = tuple(bigb for _ in range(rows_w * (gw // 2)))

            def body(jc, accs, xv_t=xv_t, gw=gw):
                j0 = jc * _LANES
                pyv = py_v[pl.ds(j0, _LANES)]
                pxv = px_v[pl.ds(j0, _LANES)]
                out = list(accs)
                for k in range(_LANES):
                    pyj = pyv[k]
                    pxj = pxv[k]
                    packed = []
                    for g in range(gw // 2):
                        dxa = xv_t[2 * g] - pxj
                        dxb = xv_t[2 * g + 1] - pxj
                        packed.append(plsc.pack(dxa * dxa, dxb * dxb,
                                                format=plsc.PackFormat.INTERLEAVED))
                    for r in range(rows_w):
                        dy = yfs[r] - pyj
                        dy2v = jnp.broadcast_to(dy * dy, (_LANES,))
                        dy2b = plsc.pack(dy2v, dy2v,
                                         format=plsc.PackFormat.INTERLEAVED)
                        for g in range(gw // 2):
                            i = r * (gw // 2) + g
                            out[i] = jnp.minimum(out[i], packed[g] + dy2b)
                return tuple(out)

            accs = lax.fori_loop(0, chunks, body, init)
            for a in accs:
                fa, fb = plsc.unpack(a, format=plsc.PackFormat.INTERLEAVED,
                                     preferred_element_type=jnp.float32)
                sum_vec = sum_vec + _sqrt_vec(fa) + _sqrt_vec(fb)
            g0 += gw

        total = jnp.sum(sum_vec)
        part_v[...] = jnp.broadcast_to(total, (_LANES,))
        pltpu.sync_copy(part_v, out.at[wid])

    return sc_k


@functools.lru_cache(maxsize=None)
def _tc_band_fn(h, w, n, h_sc):
    rows_tc = h - h_sc
    n_strips = rows_tc // 8
    assert n_strips * 8 == rows_tc
    n_cols = (w + 127) // 128

    def tc_k(py_s, px_s, out_ref):
        yv0 = lax.broadcasted_iota(jnp.int32, (8, 128), 0).astype(jnp.float32)
        xv0 = lax.broadcasted_iota(jnp.int32, (8, 128), 1).astype(jnp.float32)
        xvs = [xv0 + float(c * 128) for c in range(n_cols)]
        big = jnp.full((8, 128), 1e30, jnp.float32)
        init = tuple(big for _ in range(n_strips * n_cols))

        def body(j, accs):
            pyj = py_s[j]
            pxj = px_s[j]
            dx2 = []
            for c in range(n_cols):
                dx = xvs[c] - pxj
                dx2.append(dx * dx)
            out = list(accs)
            for s in range(n_strips):
                dy = yv0 - (pyj - float(h_sc + 8 * s))
                dy2 = dy * dy
                for c in range(n_cols):
                    i = s * n_cols + c
                    out[i] = jnp.minimum(out[i], dx2[c] + dy2)
            return tuple(out)

        accs = lax.fori_loop(0, n, body, init)
        total = jnp.float32(0.0)
        for s in range(n_strips):
            for c in range(n_cols):
                d = jnp.sqrt(accs[s * n_cols + c])
                if (c + 1) * 128 > w:
                    d = jnp.where(xvs[c] < float(w), d, 0.0)
                total = total + jnp.sum(d)
        out_ref[...] = jnp.broadcast_to(total, (1, 1))

    return tc_k


def kernel(img_render_points, img_ref):
    pts = img_render_points.reshape(-1, 2).astype(jnp.float32)
    n = pts.shape[0]
    h, w = img_ref.shape
    py = pts[:, 0]
    px = pts[:, 1]

    sc_partials = _sc_chamfer(h, w, n, _H_SC)(py, px)

    tc_out = pl.pallas_call(
        _tc_band_fn(h, w, n, _H_SC),
        in_specs=[
            pl.BlockSpec(memory_space=pltpu.SMEM),
            pl.BlockSpec(memory_space=pltpu.SMEM),
        ],
        out_shape=jax.ShapeDtypeStruct((1, 1), jnp.float32),
    )(py, px)

    return jnp.sum(sc_partials[:, 0]) + tc_out[0, 0]
